# baseline (speedup 1.0000x reference)
import jax
import jax.numpy as jnp
from jax import lax
from jax.experimental import pallas as pl
from jax.experimental.pallas import tpu as pltpu
from jax.experimental.pallas import tpu_sc as plsc

_N = 10000
_E = 320000
_H = 128

_NC = 2
_NS = 16
_NW = _NC * _NS


_PREP_BLK = 2000


def _dot(a, b):
    return lax.dot_general(a, b, (((1,), (0,)), ((), ())),
                           preferred_element_type=jnp.float32)


def _prep_body(x_ref, pe_ref, W1_ref, b1_ref, Wp1_ref, bp1_ref,
               ts_ref, tr_ref):
    H = _H
    x = x_ref[...]
    pe = pe_ref[...]
    bf = jnp.bfloat16
    ts_ref[:, 0:H] = (_dot(x, W1_ref[0:H]) + _dot(pe, W1_ref[H:2 * H])
                      + b1_ref[...]).astype(bf)
    ts_ref[:, H:2 * H] = (_dot(pe, Wp1_ref[0:H]) + bp1_ref[...]).astype(bf)
    tr_ref[:, 0:H] = (_dot(x, W1_ref[2 * H:3 * H])
                      + _dot(pe, W1_ref[3 * H:4 * H])).astype(bf)
    tr_ref[:, H:2 * H] = _dot(pe, Wp1_ref[H:2 * H]).astype(bf)


_prep = pl.pallas_call(
    _prep_body,
    grid=(_N // _PREP_BLK,),
    in_specs=[
        pl.BlockSpec((_PREP_BLK, _H), lambda i: (i, 0)),
        pl.BlockSpec((_PREP_BLK, _H), lambda i: (i, 0)),
        pl.BlockSpec((4 * _H + 1, _H), lambda i: (0, 0)),
        pl.BlockSpec((1, _H), lambda i: (0, 0)),
        pl.BlockSpec((2 * _H + 1, _H), lambda i: (0, 0)),
        pl.BlockSpec((1, _H), lambda i: (0, 0)),
    ],
    out_specs=[
        pl.BlockSpec((_PREP_BLK, 2 * _H), lambda i: (i, 0)),
        pl.BlockSpec((_PREP_BLK, 2 * _H), lambda i: (i, 0)),
    ],
    out_shape=[
        jax.ShapeDtypeStruct((_N, 2 * _H), jnp.bfloat16),
        jax.ShapeDtypeStruct((_N, 2 * _H), jnp.bfloat16),
    ],
)


_EPW = _E // _NW
_GCH = 80
_GNCH = _EPW // _GCH


def _gather_body(ts_hbm, tr_hbm, posT_hbm, send_hbm, rec_hbm,
                 pre_hbm, d2_hbm,
                 sidx_a, ridx_a, rs_a, rr_a,
                 sidx_b, ridx_b, rs_b, rr_b,
                 posT_v, d2_all,
                 isem_a, isem_b, g1sem_a, g2sem_a, g1sem_b, g2sem_b,
                 osem_a, osem_b):
    cid = lax.axis_index("c")
    sid = lax.axis_index("s")
    wid = sid * _NC + cid
    tbase = wid * _EPW
    pltpu.sync_copy(posT_hbm, posT_v)
    offy = jnp.full((16,), _N, jnp.int32)
    offz = jnp.full((16,), 2 * _N, jnp.int32)

    A = (sidx_a, ridx_a, rs_a, rr_a, isem_a, g1sem_a, g2sem_a, osem_a)
    B = (sidx_b, ridx_b, rs_b, rr_b, isem_b, g1sem_b, g2sem_b, osem_b)

    def fire_idx(c, S):
        sidx, ridx = S[0], S[1]
        base = tbase + c * _GCH
        pltpu.async_copy(send_hbm.at[pl.ds(base, _GCH)], sidx, S[4])
        pltpu.async_copy(rec_hbm.at[pl.ds(base, _GCH)], ridx, S[4])

    def wait_idx(S):
        pltpu.make_async_copy(send_hbm.at[pl.ds(0, _GCH)], S[0], S[4]).wait()
        pltpu.make_async_copy(rec_hbm.at[pl.ds(0, _GCH)], S[1], S[4]).wait()

    def fire_gathers(S):
        pltpu.async_copy(ts_hbm.at[S[0]], S[2], S[5])
        pltpu.async_copy(tr_hbm.at[S[1]], S[3], S[6])

    def wait_gathers(S):
        pltpu.make_async_copy(ts_hbm.at[S[0]], S[2], S[5]).wait()
        pltpu.make_async_copy(tr_hbm.at[S[1]], S[3], S[6]).wait()

    def fire_out(c, S):
        pltpu.async_copy(S[2], pre_hbm.at[pl.ds(tbase + c * _GCH, _GCH)],
                         S[7])

    def wait_out(S):
        pltpu.make_async_copy(S[2], pre_hbm.at[pl.ds(0, _GCH)], S[7]).wait()

    def d2_compute(c, S):
        sidx, ridx = S[0], S[1]
        for g in range(_GCH // 16):
            is_ = sidx[pl.ds(g * 16, 16)]
            ir_ = ridx[pl.ds(g * 16, 16)]
            dx = (plsc.load_gather(posT_v, [is_])
                  - plsc.load_gather(posT_v, [ir_]))
            dy = (plsc.load_gather(posT_v, [is_ + offy])
                  - plsc.load_gather(posT_v, [ir_ + offy]))
            dz = (plsc.load_gather(posT_v, [is_ + offz])
                  - plsc.load_gather(posT_v, [ir_ + offz]))
            d2_all[pl.ds(c * _GCH + g * 16, 16)] = dx * dx + dy * dy + dz * dz

    def adds(S):
        rs, rr = S[2], S[3]

        def row(i, carry):
            for j in range(_H // 16):
                sl = pl.ds(j * 16, 16)
                a = plsc.bitcast(rs[i, sl], jnp.bfloat16)
                b = plsc.bitcast(rr[i, sl], jnp.bfloat16)
                rs[i, sl] = plsc.bitcast(a + b, jnp.int32)
            return carry

        lax.fori_loop(0, _GCH, row, 0)

    def half(c, S, Sp, fire_next=True):
        wait_gathers(S)
        d2_compute(c, S)

        @pl.when(c + 2 < _GNCH)
        def _():
            fire_idx(c + 2, S)

        if fire_next:
            wait_idx(Sp)

            @pl.when(c >= 1)
            def _():
                wait_out(Sp)

            fire_gathers(Sp)
        adds(S)
        fire_out(c, S)

    fire_idx(0, A)
    fire_idx(1, B)
    wait_idx(A)
    fire_gathers(A)

    def round_(g, carry):
        half(2 * g, A, B)
        half(2 * g + 1, B, A)
        return carry

    lax.fori_loop(0, _GNCH // 2, round_, 0)
    half(_GNCH - 1, A, B, fire_next=False)
    wait_out(B)
    wait_out(A)
    pltpu.sync_copy(d2_all, d2_hbm.at[pl.ds(tbase, _EPW)])


_gather = pl.kernel(
    _gather_body,
    out_type=[
        jax.ShapeDtypeStruct((_E, _H), jnp.int32),
        jax.ShapeDtypeStruct((_E,), jnp.float32),
    ],
    mesh=plsc.VectorSubcoreMesh(core_axis_name="c", subcore_axis_name="s"),
    compiler_params=pltpu.CompilerParams(needs_layout_passes=False),
    scratch_types=[
        pltpu.VMEM((_GCH,), jnp.int32),
        pltpu.VMEM((_GCH,), jnp.int32),
        pltpu.VMEM((_GCH, _H), jnp.int32),
        pltpu.VMEM((_GCH, _H), jnp.int32),
        pltpu.VMEM((_GCH,), jnp.int32),
        pltpu.VMEM((_GCH,), jnp.int32),
        pltpu.VMEM((_GCH, _H), jnp.int32),
        pltpu.VMEM((_GCH, _H), jnp.int32),
        pltpu.VMEM((3 * _N,), jnp.float32),
        pltpu.VMEM((_EPW,), jnp.float32),
        pltpu.SemaphoreType.DMA,
        pltpu.SemaphoreType.DMA,
        pltpu.SemaphoreType.DMA,
        pltpu.SemaphoreType.DMA,
        pltpu.SemaphoreType.DMA,
        pltpu.SemaphoreType.DMA,
        pltpu.SemaphoreType.DMA,
        pltpu.SemaphoreType.DMA,
    ],
)


_MLP_BLK = 2000


def _mlp_body(pre_ref, d2_ref, W2_ref, b2_ref, Wp2_ref, bp2_ref,
              wl_ref, wpl_ref, out_ref):
    H = _H
    dist = jnp.sqrt(d2_ref[...])
    h = pre_ref[:, 0:H].astype(jnp.float32) + dist * wl_ref[...]
    h = h * jax.nn.sigmoid(h)
    m = _dot(h, W2_ref[...]) + b2_ref[...]
    out_ref[0] = m * jax.nn.sigmoid(m)
    hp = jnp.tanh(pre_ref[:, H:2 * H].astype(jnp.float32)
                  + dist * wpl_ref[...])
    out_ref[1] = jnp.tanh(_dot(hp, Wp2_ref[...]) + bp2_ref[...])


_mlp = pl.pallas_call(
    _mlp_body,
    grid=(_E // _MLP_BLK,),
    in_specs=[
        pl.BlockSpec((_MLP_BLK, 2 * _H), lambda i: (i, 0)),
        pl.BlockSpec((_MLP_BLK, 1), lambda i: (i, 0)),
        pl.BlockSpec((_H, _H), lambda i: (0, 0)),
        pl.BlockSpec((1, _H), lambda i: (0, 0)),
        pl.BlockSpec((_H, _H), lambda i: (0, 0)),
        pl.BlockSpec((1, _H), lambda i: (0, 0)),
        pl.BlockSpec((1, _H), lambda i: (0, 0)),
        pl.BlockSpec((1, _H), lambda i: (0, 0)),
    ],
    out_specs=pl.BlockSpec((2, _MLP_BLK, _H), lambda i: (0, i, 0)),
    out_shape=jax.ShapeDtypeStruct((2, _E, _H), jnp.float32),
)


_NP = 2
_NH = _N // _NP
_SROWS = _NH + 8
_NPT = 312
_NTAIL = _NH - _NS * _NPT
_EPT = _E // _NS
_SCH = 80
_SNCH = _EPT // _SCH
_NBUF = 4


def _scatter_body(msgs_hbm, init_hbm, rec_hbm, out_hbm,
                  idxb0, idxb1, idxb2, idxb3,
                  cidx0, cidx1, cidx2, cidx3,
                  mbuf0, mbuf1, mbuf2, mbuf3,
                  shared,
                  msem0, msem1, msem2, msem3,
                  ssem0, ssem1, ssem2, ssem3):
    cid = lax.axis_index("c")
    sid = lax.axis_index("s")
    row0 = pl.multiple_of(sid * _NPT, 8)
    tail0 = _NS * _NPT
    ebase = sid * _EPT
    idxb = (idxb0, idxb1, idxb2, idxb3)
    cidx = (cidx0, cidx1, cidx2, cidx3)
    mbuf = (mbuf0, mbuf1, mbuf2, mbuf3)
    msem = (msem0, msem1, msem2, msem3)
    ssem = (ssem0, ssem1, ssem2, ssem3)

    for p in range(_NP):
        nbase = p * _NH
        pltpu.sync_copy(init_hbm.at[cid, pl.ds(nbase + row0, _NPT)],
                        shared.at[pl.ds(row0, _NPT)])

        @pl.when(sid == _NS - 1)
        def _():
            pltpu.sync_copy(init_hbm.at[cid, pl.ds(nbase + tail0, _NTAIL)],
                            shared.at[pl.ds(tail0, _NTAIL)])

        plsc.subcore_barrier()

        def build_cidx(b):
            for g in range(_SCH // 16):
                v = idxb[b][pl.ds(g * 16, 16)] - (p * _NH)
                ok = (v >= 0) & (v < _NH)
                grow = (lax.iota(jnp.int32, 16) & 7) + _NH
                cidx[b][pl.ds(g * 16, 16)] = jnp.where(ok, v, grow)

        def fire_msg(c, b):
            pltpu.async_copy(
                msgs_hbm.at[cid, pl.ds(ebase + c * _SCH, _SCH)],
                mbuf[b], msem[b])
            pltpu.async_copy(rec_hbm.at[pl.ds(ebase + c * _SCH, _SCH)],
                             idxb[b], msem[b])

        def wait_msg(b):
            pltpu.make_async_copy(
                msgs_hbm.at[cid, pl.ds(0, _SCH)], mbuf[b], msem[b]).wait()
            pltpu.make_async_copy(rec_hbm.at[pl.ds(0, _SCH)], idxb[b],
                                  msem[b]).wait()

        def fire_sc(b):
            pltpu.async_copy(mbuf[b], shared.at[cidx[b]], ssem[b], add=True)

        def wait_sc(b):
            pltpu.make_async_copy(mbuf[b], shared.at[cidx[b]],
                                  ssem[b]).wait()

        def turn(c, b):
            wait_msg(b)
            build_cidx(b)
            fire_sc(b)
            b2 = (b + 2) % _NBUF

            @pl.when(c >= 2)
            def _():
                wait_sc(b2)

            @pl.when(c + 2 < _SNCH)
            def _():
                fire_msg(c + 2, b2)

        fire_msg(0, 0)
        fire_msg(1, 1)

        def round_(g, carry):
            for b in range(_NBUF):
                turn(_NBUF * g + b, b)
            return carry

        lax.fori_loop(0, _SNCH // _NBUF, round_, 0)
        turn(_SNCH - 2, 0)
        turn(_SNCH - 1, 1)
        wait_sc(0)
        wait_sc(1)

        plsc.subcore_barrier()
        pltpu.sync_copy(shared.at[pl.ds(row0, _NPT)],
                        out_hbm.at[cid, pl.ds(nbase + row0, _NPT)])

        @pl.when(sid == _NS - 1)
        def _():
            pltpu.sync_copy(shared.at[pl.ds(tail0, _NTAIL)],
                            out_hbm.at[cid, pl.ds(nbase + tail0, _NTAIL)])

        plsc.subcore_barrier()


_scatter = pl.kernel(
    _scatter_body,
    out_type=jax.ShapeDtypeStruct((2, _N, _H), jnp.float32),
    mesh=plsc.VectorSubcoreMesh(core_axis_name="c", subcore_axis_name="s"),
    compiler_params=pltpu.CompilerParams(needs_layout_passes=False),
    scratch_types=[
        pltpu.VMEM((_SCH,), jnp.int32),
        pltpu.VMEM((_SCH,), jnp.int32),
        pltpu.VMEM((_SCH,), jnp.int32),
        pltpu.VMEM((_SCH,), jnp.int32),
        pltpu.VMEM((_SCH,), jnp.int32),
        pltpu.VMEM((_SCH,), jnp.int32),
        pltpu.VMEM((_SCH,), jnp.int32),
        pltpu.VMEM((_SCH,), jnp.int32),
        pltpu.VMEM((_SCH, _H), jnp.float32),
        pltpu.VMEM((_SCH, _H), jnp.float32),
        pltpu.VMEM((_SCH, _H), jnp.float32),
        pltpu.VMEM((_SCH, _H), jnp.float32),
        pltpu.VMEM_SHARED((_SROWS, _H), jnp.float32),
        pltpu.SemaphoreType.DMA,
        pltpu.SemaphoreType.DMA,
        pltpu.SemaphoreType.DMA,
        pltpu.SemaphoreType.DMA,
        pltpu.SemaphoreType.DMA,
        pltpu.SemaphoreType.DMA,
        pltpu.SemaphoreType.DMA,
        pltpu.SemaphoreType.DMA,
    ],
)


def kernel(x, pos, pe, edge_index, W1, b1, W2, b2, Wp1, bp1, Wp2, bp2):
    send = edge_index[0]
    rec = edge_index[1]
    posT = pos.T.reshape(3 * _N)
    ts, tr = _prep(x, pe, W1, b1.reshape(1, _H), Wp1, bp1.reshape(1, _H))
    ts32 = lax.bitcast_convert_type(ts.reshape(_N, _H, 2), jnp.int32)
    tr32 = lax.bitcast_convert_type(tr.reshape(_N, _H, 2), jnp.int32)
    pre32, d2 = _gather(ts32, tr32, posT, send, rec)
    pre = lax.bitcast_convert_type(pre32, jnp.bfloat16).reshape(_E, 2 * _H)
    msgs = _mlp(pre, d2[:, None], W2, b2.reshape(1, _H), Wp2,
                bp2.reshape(1, _H), W1[4 * _H:4 * _H + 1],
                Wp1[2 * _H:2 * _H + 1])
    init = jnp.stack([x, pe])
    out = _scatter(msgs, init, rec)
    return out[0], out[1]


# speedup vs baseline: 2.3105x; 2.3105x over previous
import jax
import jax.numpy as jnp
from jax import lax
from jax.experimental import pallas as pl
from jax.experimental.pallas import tpu as pltpu
from jax.experimental.pallas import tpu_sc as plsc

_N = 10000
_E = 320000
_H = 128

_NC = 2
_NS = 16
_NW = _NC * _NS


_PREP_BLK = 2000


def _dot(a, b):
    return lax.dot_general(a, b, (((1,), (0,)), ((), ())),
                           preferred_element_type=jnp.float32)


def _pack(a, b):
    ua = lax.bitcast_convert_type(
        a.astype(jnp.bfloat16).astype(jnp.float32), jnp.uint32)
    ub = lax.bitcast_convert_type(
        b.astype(jnp.bfloat16).astype(jnp.float32), jnp.uint32)
    w = (ua >> 16) | (ub & jnp.uint32(0xFFFF0000))
    return lax.bitcast_convert_type(w, jnp.int32)


def _prep_body(x_ref, pe_ref, W1_ref, b1_ref, Wp1_ref, bp1_ref,
               ts_ref, tr_ref):
    H = _H
    Q = _H // 2
    x = x_ref[...]
    pe = pe_ref[...]
    A = _dot(x, W1_ref[0:H]) + _dot(pe, W1_ref[H:2 * H]) + b1_ref[...]
    Ap = _dot(pe, Wp1_ref[0:H]) + bp1_ref[...]
    Bm = _dot(x, W1_ref[2 * H:3 * H]) + _dot(pe, W1_ref[3 * H:4 * H])
    Bp = _dot(pe, Wp1_ref[H:2 * H])
    ts_ref[:, 0:Q] = _pack(A[:, 0:Q], A[:, Q:H])
    ts_ref[:, Q:H] = _pack(Ap[:, 0:Q], Ap[:, Q:H])
    tr_ref[:, 0:Q] = _pack(Bm[:, 0:Q], Bm[:, Q:H])
    tr_ref[:, Q:H] = _pack(Bp[:, 0:Q], Bp[:, Q:H])


_prep = pl.pallas_call(
    _prep_body,
    grid=(_N // _PREP_BLK,),
    in_specs=[
        pl.BlockSpec((_PREP_BLK, _H), lambda i: (i, 0)),
        pl.BlockSpec((_PREP_BLK, _H), lambda i: (i, 0)),
        pl.BlockSpec((4 * _H + 1, _H), lambda i: (0, 0)),
        pl.BlockSpec((1, _H), lambda i: (0, 0)),
        pl.BlockSpec((2 * _H + 1, _H), lambda i: (0, 0)),
        pl.BlockSpec((1, _H), lambda i: (0, 0)),
    ],
    out_specs=[
        pl.BlockSpec((_PREP_BLK, _H), lambda i: (i, 0)),
        pl.BlockSpec((_PREP_BLK, _H), lambda i: (i, 0)),
    ],
    out_shape=[
        jax.ShapeDtypeStruct((_N, _H), jnp.int32),
        jax.ShapeDtypeStruct((_N, _H), jnp.int32),
    ],
)


_EPW = _E // _NW
_GCH = 80
_GNCH = _EPW // _GCH


def _gather_body(ts_hbm, tr_hbm, posT_hbm, send_hbm, rec_hbm,
                 pre_hbm, d2_hbm,
                 sidx_a, ridx_a, rs_a, rr_a,
                 sidx_b, ridx_b, rs_b, rr_b,
                 posT_v, d2_all,
                 isem_a, isem_b, g1sem_a, g2sem_a, g1sem_b, g2sem_b,
                 osem_a, osem_b):
    cid = lax.axis_index("c")
    sid = lax.axis_index("s")
    wid = sid * _NC + cid
    tbase = wid * _EPW
    pltpu.sync_copy(posT_hbm, posT_v)
    offy = jnp.full((16,), _N, jnp.int32)
    offz = jnp.full((16,), 2 * _N, jnp.int32)

    A = (sidx_a, ridx_a, rs_a, rr_a, isem_a, g1sem_a, g2sem_a, osem_a)
    B = (sidx_b, ridx_b, rs_b, rr_b, isem_b, g1sem_b, g2sem_b, osem_b)

    def fire_idx(c, S):
        sidx, ridx = S[0], S[1]
        base = tbase + c * _GCH
        pltpu.async_copy(send_hbm.at[pl.ds(base, _GCH)], sidx, S[4])
        pltpu.async_copy(rec_hbm.at[pl.ds(base, _GCH)], ridx, S[4])

    def wait_idx(S):
        pltpu.make_async_copy(send_hbm.at[pl.ds(0, _GCH)], S[0], S[4]).wait()
        pltpu.make_async_copy(rec_hbm.at[pl.ds(0, _GCH)], S[1], S[4]).wait()

    def fire_gathers(S):
        pltpu.async_copy(ts_hbm.at[S[0]], S[2], S[5])
        pltpu.async_copy(tr_hbm.at[S[1]], S[3], S[6])

    def wait_gathers(S):
        pltpu.make_async_copy(ts_hbm.at[S[0]], S[2], S[5]).wait()
        pltpu.make_async_copy(tr_hbm.at[S[1]], S[3], S[6]).wait()

    def fire_out(c, S):
        pltpu.async_copy(S[2], pre_hbm.at[pl.ds(tbase + c * _GCH, _GCH)],
                         S[7])

    def wait_out(S):
        pltpu.make_async_copy(S[2], pre_hbm.at[pl.ds(0, _GCH)], S[7]).wait()

    def d2_compute(c, S):
        sidx, ridx = S[0], S[1]
        for g in range(_GCH // 16):
            is_ = sidx[pl.ds(g * 16, 16)]
            ir_ = ridx[pl.ds(g * 16, 16)]
            dx = (plsc.load_gather(posT_v, [is_])
                  - plsc.load_gather(posT_v, [ir_]))
            dy = (plsc.load_gather(posT_v, [is_ + offy])
                  - plsc.load_gather(posT_v, [ir_ + offy]))
            dz = (plsc.load_gather(posT_v, [is_ + offz])
                  - plsc.load_gather(posT_v, [ir_ + offz]))
            d2_all[pl.ds(c * _GCH + g * 16, 16)] = dx * dx + dy * dy + dz * dz

    def adds(S):
        rs, rr = S[2], S[3]

        def row(i, carry):
            for j in range(_H // 16):
                sl = pl.ds(j * 16, 16)
                a = plsc.bitcast(rs[i, sl], jnp.bfloat16)
                b = plsc.bitcast(rr[i, sl], jnp.bfloat16)
                rs[i, sl] = plsc.bitcast(a + b, jnp.int32)
            return carry

        lax.fori_loop(0, _GCH, row, 0)

    def half(c, S, Sp, fire_next=True):
        wait_gathers(S)
        d2_compute(c, S)

        @pl.when(c + 2 < _GNCH)
        def _():
            fire_idx(c + 2, S)

        if fire_next:
            wait_idx(Sp)

            @pl.when(c >= 1)
            def _():
                wait_out(Sp)

            fire_gathers(Sp)
        adds(S)
        fire_out(c, S)

    fire_idx(0, A)
    fire_idx(1, B)
    wait_idx(A)
    fire_gathers(A)

    def round_(g, carry):
        half(2 * g, A, B)
        half(2 * g + 1, B, A)
        return carry

    lax.fori_loop(0, _GNCH // 2, round_, 0)
    half(_GNCH - 1, A, B, fire_next=False)
    wait_out(B)
    wait_out(A)
    pltpu.sync_copy(d2_all, d2_hbm.at[pl.ds(tbase, _EPW)])


_gather = pl.kernel(
    _gather_body,
    out_type=[
        jax.ShapeDtypeStruct((_E, _H), jnp.int32),
        jax.ShapeDtypeStruct((_E,), jnp.float32),
    ],
    mesh=plsc.VectorSubcoreMesh(core_axis_name="c", subcore_axis_name="s"),
    compiler_params=pltpu.CompilerParams(needs_layout_passes=False),
    scratch_types=[
        pltpu.VMEM((_GCH,), jnp.int32),
        pltpu.VMEM((_GCH,), jnp.int32),
        pltpu.VMEM((_GCH, _H), jnp.int32),
        pltpu.VMEM((_GCH, _H), jnp.int32),
        pltpu.VMEM((_GCH,), jnp.int32),
        pltpu.VMEM((_GCH,), jnp.int32),
        pltpu.VMEM((_GCH, _H), jnp.int32),
        pltpu.VMEM((_GCH, _H), jnp.int32),
        pltpu.VMEM((3 * _N,), jnp.float32),
        pltpu.VMEM((_EPW,), jnp.float32),
        pltpu.SemaphoreType.DMA,
        pltpu.SemaphoreType.DMA,
        pltpu.SemaphoreType.DMA,
        pltpu.SemaphoreType.DMA,
        pltpu.SemaphoreType.DMA,
        pltpu.SemaphoreType.DMA,
        pltpu.SemaphoreType.DMA,
        pltpu.SemaphoreType.DMA,
    ],
)


_MLP_BLK = 2000


def _unpack(w):
    u = lax.bitcast_convert_type(w, jnp.uint32)
    lo = lax.bitcast_convert_type(u << 16, jnp.float32)
    hi = lax.bitcast_convert_type(u & jnp.uint32(0xFFFF0000), jnp.float32)
    return jnp.concatenate([lo, hi], axis=1)


def _mlp_body(pre_ref, d2_ref, W2_ref, b2_ref, Wp2_ref, bp2_ref,
              wl_ref, wpl_ref, out_ref):
    H = _H
    Q = _H // 2
    dist = jnp.sqrt(d2_ref[...])
    h = _unpack(pre_ref[:, 0:Q]) + dist * wl_ref[...]
    h = h * jax.nn.sigmoid(h)
    m = _dot(h, W2_ref[...]) + b2_ref[...]
    out_ref[0] = m * jax.nn.sigmoid(m)
    hp = jnp.tanh(_unpack(pre_ref[:, Q:H]) + dist * wpl_ref[...])
    out_ref[1] = jnp.tanh(_dot(hp, Wp2_ref[...]) + bp2_ref[...])


_mlp = pl.pallas_call(
    _mlp_body,
    grid=(_E // _MLP_BLK,),
    in_specs=[
        pl.BlockSpec((_MLP_BLK, _H), lambda i: (i, 0)),
        pl.BlockSpec((_MLP_BLK, 1), lambda i: (i, 0)),
        pl.BlockSpec((_H, _H), lambda i: (0, 0)),
        pl.BlockSpec((1, _H), lambda i: (0, 0)),
        pl.BlockSpec((_H, _H), lambda i: (0, 0)),
        pl.BlockSpec((1, _H), lambda i: (0, 0)),
        pl.BlockSpec((1, _H), lambda i: (0, 0)),
        pl.BlockSpec((1, _H), lambda i: (0, 0)),
    ],
    out_specs=pl.BlockSpec((2, _MLP_BLK, _H), lambda i: (0, i, 0)),
    out_shape=jax.ShapeDtypeStruct((2, _E, _H), jnp.float32),
)


_NP = 2
_NH = _N // _NP
_SROWS = _NH + 8
_NPT = 312
_NTAIL = _NH - _NS * _NPT
_EPT = _E // _NS
_SCH = 80
_SNCH = _EPT // _SCH
_NBUF = 4


def _scatter_body(msgs_hbm, init_hbm, rec_hbm, out_hbm,
                  idxb0, idxb1, idxb2, idxb3,
                  cidx0, cidx1, cidx2, cidx3,
                  mbuf0, mbuf1, mbuf2, mbuf3,
                  shared,
                  msem0, msem1, msem2, msem3,
                  ssem0, ssem1, ssem2, ssem3):
    cid = lax.axis_index("c")
    sid = lax.axis_index("s")
    row0 = pl.multiple_of(sid * _NPT, 8)
    tail0 = _NS * _NPT
    ebase = sid * _EPT
    idxb = (idxb0, idxb1, idxb2, idxb3)
    cidx = (cidx0, cidx1, cidx2, cidx3)
    mbuf = (mbuf0, mbuf1, mbuf2, mbuf3)
    msem = (msem0, msem1, msem2, msem3)
    ssem = (ssem0, ssem1, ssem2, ssem3)

    for p in range(_NP):
        nbase = p * _NH
        pltpu.sync_copy(init_hbm.at[cid, pl.ds(nbase + row0, _NPT)],
                        shared.at[pl.ds(row0, _NPT)])

        @pl.when(sid == _NS - 1)
        def _():
            pltpu.sync_copy(init_hbm.at[cid, pl.ds(nbase + tail0, _NTAIL)],
                            shared.at[pl.ds(tail0, _NTAIL)])

        plsc.subcore_barrier()

        def build_cidx(b):
            for g in range(_SCH // 16):
                v = idxb[b][pl.ds(g * 16, 16)] - (p * _NH)
                ok = (v >= 0) & (v < _NH)
                grow = (lax.iota(jnp.int32, 16) & 7) + _NH
                cidx[b][pl.ds(g * 16, 16)] = jnp.where(ok, v, grow)

        def fire_msg(c, b):
            pltpu.async_copy(
                msgs_hbm.at[cid, pl.ds(ebase + c * _SCH, _SCH)],
                mbuf[b], msem[b])
            pltpu.async_copy(rec_hbm.at[pl.ds(ebase + c * _SCH, _SCH)],
                             idxb[b], msem[b])

        def wait_msg(b):
            pltpu.make_async_copy(
                msgs_hbm.at[cid, pl.ds(0, _SCH)], mbuf[b], msem[b]).wait()
            pltpu.make_async_copy(rec_hbm.at[pl.ds(0, _SCH)], idxb[b],
                                  msem[b]).wait()

        def fire_sc(b):
            pltpu.async_copy(mbuf[b], shared.at[cidx[b]], ssem[b], add=True)

        def wait_sc(b):
            pltpu.make_async_copy(mbuf[b], shared.at[cidx[b]],
                                  ssem[b]).wait()

        def turn(c, b):
            wait_msg(b)
            build_cidx(b)
            fire_sc(b)
            b2 = (b + 2) % _NBUF

            @pl.when(c >= 2)
            def _():
                wait_sc(b2)

            @pl.when(c + 2 < _SNCH)
            def _():
                fire_msg(c + 2, b2)

        fire_msg(0, 0)
        fire_msg(1, 1)

        def round_(g, carry):
            for b in range(_NBUF):
                turn(_NBUF * g + b, b)
            return carry

        lax.fori_loop(0, _SNCH // _NBUF, round_, 0)
        turn(_SNCH - 2, 0)
        turn(_SNCH - 1, 1)
        wait_sc(0)
        wait_sc(1)

        plsc.subcore_barrier()
        pltpu.sync_copy(shared.at[pl.ds(row0, _NPT)],
                        out_hbm.at[cid, pl.ds(nbase + row0, _NPT)])

        @pl.when(sid == _NS - 1)
        def _():
            pltpu.sync_copy(shared.at[pl.ds(tail0, _NTAIL)],
                            out_hbm.at[cid, pl.ds(nbase + tail0, _NTAIL)])

        plsc.subcore_barrier()


_scatter = pl.kernel(
    _scatter_body,
    out_type=jax.ShapeDtypeStruct((2, _N, _H), jnp.float32),
    mesh=plsc.VectorSubcoreMesh(core_axis_name="c", subcore_axis_name="s"),
    compiler_params=pltpu.CompilerParams(needs_layout_passes=False),
    scratch_types=[
        pltpu.VMEM((_SCH,), jnp.int32),
        pltpu.VMEM((_SCH,), jnp.int32),
        pltpu.VMEM((_SCH,), jnp.int32),
        pltpu.VMEM((_SCH,), jnp.int32),
        pltpu.VMEM((_SCH,), jnp.int32),
        pltpu.VMEM((_SCH,), jnp.int32),
        pltpu.VMEM((_SCH,), jnp.int32),
        pltpu.VMEM((_SCH,), jnp.int32),
        pltpu.VMEM((_SCH, _H), jnp.float32),
        pltpu.VMEM((_SCH, _H), jnp.float32),
        pltpu.VMEM((_SCH, _H), jnp.float32),
        pltpu.VMEM((_SCH, _H), jnp.float32),
        pltpu.VMEM_SHARED((_SROWS, _H), jnp.float32),
        pltpu.SemaphoreType.DMA,
        pltpu.SemaphoreType.DMA,
        pltpu.SemaphoreType.DMA,
        pltpu.SemaphoreType.DMA,
        pltpu.SemaphoreType.DMA,
        pltpu.SemaphoreType.DMA,
        pltpu.SemaphoreType.DMA,
        pltpu.SemaphoreType.DMA,
    ],
)


def kernel(x, pos, pe, edge_index, W1, b1, W2, b2, Wp1, bp1, Wp2, bp2):
    send = edge_index[0]
    rec = edge_index[1]
    posT = pos.T.reshape(3 * _N)
    ts32, tr32 = _prep(x, pe, W1, b1.reshape(1, _H), Wp1,
                       bp1.reshape(1, _H))
    pre32, d2 = _gather(ts32, tr32, posT, send, rec)
    msgs = _mlp(pre32, d2[:, None], W2, b2.reshape(1, _H), Wp2,
                bp2.reshape(1, _H), W1[4 * _H:4 * _H + 1],
                Wp1[2 * _H:2 * _H + 1])
    init = jnp.stack([x, pe])
    out = _scatter(msgs, init, rec)
    return out[0], out[1]


# speedup vs baseline: 2.3430x; 1.0141x over previous
import jax
import jax.numpy as jnp
from jax import lax
from jax.experimental import pallas as pl
from jax.experimental.pallas import tpu as pltpu
from jax.experimental.pallas import tpu_sc as plsc

_N = 10000
_E = 320000
_H = 128

_NC = 2
_NS = 16
_NW = _NC * _NS


_PREP_BLK = 2000


def _dot(a, b):
    return lax.dot_general(a, b, (((1,), (0,)), ((), ())),
                           preferred_element_type=jnp.float32)


def _pack(a, b):
    ua = lax.bitcast_convert_type(
        a.astype(jnp.bfloat16).astype(jnp.float32), jnp.uint32)
    ub = lax.bitcast_convert_type(
        b.astype(jnp.bfloat16).astype(jnp.float32), jnp.uint32)
    w = (ua >> 16) | (ub & jnp.uint32(0xFFFF0000))
    return lax.bitcast_convert_type(w, jnp.int32)


def _prep_body(x_ref, pe_ref, W1_ref, b1_ref, Wp1_ref, bp1_ref,
               ts_ref, tr_ref):
    H = _H
    Q = _H // 2
    x = x_ref[...]
    pe = pe_ref[...]
    A = _dot(x, W1_ref[0:H]) + _dot(pe, W1_ref[H:2 * H]) + b1_ref[...]
    Ap = _dot(pe, Wp1_ref[0:H]) + bp1_ref[...]
    Bm = _dot(x, W1_ref[2 * H:3 * H]) + _dot(pe, W1_ref[3 * H:4 * H])
    Bp = _dot(pe, Wp1_ref[H:2 * H])
    ts_ref[:, 0:Q] = _pack(A[:, 0:Q], A[:, Q:H])
    ts_ref[:, Q:H] = _pack(Ap[:, 0:Q], Ap[:, Q:H])
    tr_ref[:, 0:Q] = _pack(Bm[:, 0:Q], Bm[:, Q:H])
    tr_ref[:, Q:H] = _pack(Bp[:, 0:Q], Bp[:, Q:H])


_prep = pl.pallas_call(
    _prep_body,
    grid=(_N // _PREP_BLK,),
    in_specs=[
        pl.BlockSpec((_PREP_BLK, _H), lambda i: (i, 0)),
        pl.BlockSpec((_PREP_BLK, _H), lambda i: (i, 0)),
        pl.BlockSpec((4 * _H + 1, _H), lambda i: (0, 0)),
        pl.BlockSpec((1, _H), lambda i: (0, 0)),
        pl.BlockSpec((2 * _H + 1, _H), lambda i: (0, 0)),
        pl.BlockSpec((1, _H), lambda i: (0, 0)),
    ],
    out_specs=[
        pl.BlockSpec((_PREP_BLK, _H), lambda i: (i, 0)),
        pl.BlockSpec((_PREP_BLK, _H), lambda i: (i, 0)),
    ],
    out_shape=[
        jax.ShapeDtypeStruct((_N, _H), jnp.int32),
        jax.ShapeDtypeStruct((_N, _H), jnp.int32),
    ],
)


_EPW = _E // _NW
_GCH = 80
_GNCH = _EPW // _GCH


def _gather_body(ts_hbm, tr_hbm, posT_hbm, send_hbm, rec_hbm,
                 pre_hbm, d2_hbm,
                 sidx_a, ridx_a, rs_a, rr_a,
                 sidx_b, ridx_b, rs_b, rr_b,
                 posT_v, d2_all,
                 isem_a, isem_b, g1sem_a, g2sem_a, g1sem_b, g2sem_b,
                 osem_a, osem_b):
    cid = lax.axis_index("c")
    sid = lax.axis_index("s")
    wid = sid * _NC + cid
    tbase = wid * _EPW
    pltpu.sync_copy(posT_hbm, posT_v)
    offy = jnp.full((16,), _N, jnp.int32)
    offz = jnp.full((16,), 2 * _N, jnp.int32)

    A = (sidx_a, ridx_a, rs_a, rr_a, isem_a, g1sem_a, g2sem_a, osem_a)
    B = (sidx_b, ridx_b, rs_b, rr_b, isem_b, g1sem_b, g2sem_b, osem_b)

    def fire_idx(c, S):
        sidx, ridx = S[0], S[1]
        base = tbase + c * _GCH
        pltpu.async_copy(send_hbm.at[pl.ds(base, _GCH)], sidx, S[4])
        pltpu.async_copy(rec_hbm.at[pl.ds(base, _GCH)], ridx, S[4])

    def wait_idx(S):
        pltpu.make_async_copy(send_hbm.at[pl.ds(0, _GCH)], S[0], S[4]).wait()
        pltpu.make_async_copy(rec_hbm.at[pl.ds(0, _GCH)], S[1], S[4]).wait()

    def fire_gathers(S):
        pltpu.async_copy(ts_hbm.at[S[0]], S[2], S[5])
        pltpu.async_copy(tr_hbm.at[S[1]], S[3], S[6])

    def wait_gathers(S):
        pltpu.make_async_copy(ts_hbm.at[S[0]], S[2], S[5]).wait()
        pltpu.make_async_copy(tr_hbm.at[S[1]], S[3], S[6]).wait()

    def fire_out(c, S):
        pltpu.async_copy(S[2], pre_hbm.at[pl.ds(tbase + c * _GCH, _GCH)],
                         S[7])

    def wait_out(S):
        pltpu.make_async_copy(S[2], pre_hbm.at[pl.ds(0, _GCH)], S[7]).wait()

    def d2_compute(c, S):
        sidx, ridx = S[0], S[1]
        for g in range(_GCH // 16):
            is_ = sidx[pl.ds(g * 16, 16)]
            ir_ = ridx[pl.ds(g * 16, 16)]
            dx = (plsc.load_gather(posT_v, [is_])
                  - plsc.load_gather(posT_v, [ir_]))
            dy = (plsc.load_gather(posT_v, [is_ + offy])
                  - plsc.load_gather(posT_v, [ir_ + offy]))
            dz = (plsc.load_gather(posT_v, [is_ + offz])
                  - plsc.load_gather(posT_v, [ir_ + offz]))
            d2_all[pl.ds(c * _GCH + g * 16, 16)] = dx * dx + dy * dy + dz * dz

    def adds(S):
        rs, rr = S[2], S[3]

        def row(i, carry):
            for j in range(_H // 16):
                sl = pl.ds(j * 16, 16)
                a = plsc.bitcast(rs[i, sl], jnp.bfloat16)
                b = plsc.bitcast(rr[i, sl], jnp.bfloat16)
                rs[i, sl] = plsc.bitcast(a + b, jnp.int32)
            return carry

        lax.fori_loop(0, _GCH, row, 0)

    def half(c, S, Sp, fire_next=True):
        wait_gathers(S)
        d2_compute(c, S)

        @pl.when(c + 2 < _GNCH)
        def _():
            fire_idx(c + 2, S)

        if fire_next:
            wait_idx(Sp)

            @pl.when(c >= 1)
            def _():
                wait_out(Sp)

            fire_gathers(Sp)
        adds(S)
        fire_out(c, S)

    fire_idx(0, A)
    fire_idx(1, B)
    wait_idx(A)
    fire_gathers(A)

    def round_(g, carry):
        half(2 * g, A, B)
        half(2 * g + 1, B, A)
        return carry

    lax.fori_loop(0, _GNCH // 2, round_, 0)
    half(_GNCH - 1, A, B, fire_next=False)
    wait_out(B)
    wait_out(A)
    pltpu.sync_copy(d2_all, d2_hbm.at[pl.ds(tbase, _EPW)])


_gather = pl.kernel(
    _gather_body,
    out_type=[
        jax.ShapeDtypeStruct((_E, _H), jnp.int32),
        jax.ShapeDtypeStruct((_E,), jnp.float32),
    ],
    mesh=plsc.VectorSubcoreMesh(core_axis_name="c", subcore_axis_name="s"),
    compiler_params=pltpu.CompilerParams(needs_layout_passes=False),
    scratch_types=[
        pltpu.VMEM((_GCH,), jnp.int32),
        pltpu.VMEM((_GCH,), jnp.int32),
        pltpu.VMEM((_GCH, _H), jnp.int32),
        pltpu.VMEM((_GCH, _H), jnp.int32),
        pltpu.VMEM((_GCH,), jnp.int32),
        pltpu.VMEM((_GCH,), jnp.int32),
        pltpu.VMEM((_GCH, _H), jnp.int32),
        pltpu.VMEM((_GCH, _H), jnp.int32),
        pltpu.VMEM((3 * _N,), jnp.float32),
        pltpu.VMEM((_EPW,), jnp.float32),
        pltpu.SemaphoreType.DMA,
        pltpu.SemaphoreType.DMA,
        pltpu.SemaphoreType.DMA,
        pltpu.SemaphoreType.DMA,
        pltpu.SemaphoreType.DMA,
        pltpu.SemaphoreType.DMA,
        pltpu.SemaphoreType.DMA,
        pltpu.SemaphoreType.DMA,
    ],
)


_MLP_BLK = 2000


def _unpack(w):
    u = lax.bitcast_convert_type(w, jnp.uint32)
    lo = lax.bitcast_convert_type(u << 16, jnp.float32)
    hi = lax.bitcast_convert_type(u & jnp.uint32(0xFFFF0000), jnp.float32)
    return jnp.concatenate([lo, hi], axis=1)


def _silu(x):
    return x * (0.5 * jnp.tanh(0.5 * x) + 0.5)


def _mlp_body(pre_ref, d2_ref, W2_ref, b2_ref, Wp2_ref, bp2_ref,
              wl_ref, wpl_ref, out_ref):
    H = _H
    Q = _H // 2
    dist = jnp.sqrt(d2_ref[...])
    h = _unpack(pre_ref[:, 0:Q]) + dist * wl_ref[...]
    h = _silu(h)
    m = _dot(h, W2_ref[...]) + b2_ref[...]
    out_ref[0] = _silu(m)
    hp = jnp.tanh(_unpack(pre_ref[:, Q:H]) + dist * wpl_ref[...])
    out_ref[1] = jnp.tanh(_dot(hp, Wp2_ref[...]) + bp2_ref[...])


_mlp = pl.pallas_call(
    _mlp_body,
    grid=(_E // _MLP_BLK,),
    in_specs=[
        pl.BlockSpec((_MLP_BLK, _H), lambda i: (i, 0)),
        pl.BlockSpec((_MLP_BLK, 1), lambda i: (i, 0)),
        pl.BlockSpec((_H, _H), lambda i: (0, 0)),
        pl.BlockSpec((1, _H), lambda i: (0, 0)),
        pl.BlockSpec((_H, _H), lambda i: (0, 0)),
        pl.BlockSpec((1, _H), lambda i: (0, 0)),
        pl.BlockSpec((1, _H), lambda i: (0, 0)),
        pl.BlockSpec((1, _H), lambda i: (0, 0)),
    ],
    out_specs=pl.BlockSpec((2, _MLP_BLK, _H), lambda i: (0, i, 0)),
    out_shape=jax.ShapeDtypeStruct((2, _E, _H), jnp.float32),
)


_NP = 2
_NH = _N // _NP
_SROWS = _NH + 8
_NPT = 312
_NTAIL = _NH - _NS * _NPT
_EPT = _E // _NS
_SCH = 80
_SNCH = _EPT // _SCH
_NBUF = 4


def _scatter_body(msgs_hbm, x_hbm, pe_hbm, rec_hbm, outx_hbm, outp_hbm,
                  idxb0, idxb1, idxb2, idxb3,
                  cidx0, cidx1, cidx2, cidx3,
                  mbuf0, mbuf1, mbuf2, mbuf3,
                  shared,
                  msem0, msem1, msem2, msem3,
                  ssem0, ssem1, ssem2, ssem3):
    cid = lax.axis_index("c")
    sid = lax.axis_index("s")
    row0 = pl.multiple_of(sid * _NPT, 8)
    tail0 = _NS * _NPT
    ebase = sid * _EPT
    idxb = (idxb0, idxb1, idxb2, idxb3)
    cidx = (cidx0, cidx1, cidx2, cidx3)
    mbuf = (mbuf0, mbuf1, mbuf2, mbuf3)
    msem = (msem0, msem1, msem2, msem3)
    ssem = (ssem0, ssem1, ssem2, ssem3)

    for p in range(_NP):
        nbase = p * _NH

        @pl.when(cid == 0)
        def _():
            pltpu.sync_copy(x_hbm.at[pl.ds(nbase + row0, _NPT)],
                            shared.at[pl.ds(row0, _NPT)])

        @pl.when(cid == 1)
        def _():
            pltpu.sync_copy(pe_hbm.at[pl.ds(nbase + row0, _NPT)],
                            shared.at[pl.ds(row0, _NPT)])

        @pl.when(sid == _NS - 1)
        def _():
            @pl.when(cid == 0)
            def _():
                pltpu.sync_copy(x_hbm.at[pl.ds(nbase + tail0, _NTAIL)],
                                shared.at[pl.ds(tail0, _NTAIL)])

            @pl.when(cid == 1)
            def _():
                pltpu.sync_copy(pe_hbm.at[pl.ds(nbase + tail0, _NTAIL)],
                                shared.at[pl.ds(tail0, _NTAIL)])

        plsc.subcore_barrier()

        def build_cidx(b):
            for g in range(_SCH // 16):
                v = idxb[b][pl.ds(g * 16, 16)] - (p * _NH)
                ok = (v >= 0) & (v < _NH)
                grow = (lax.iota(jnp.int32, 16) & 7) + _NH
                cidx[b][pl.ds(g * 16, 16)] = jnp.where(ok, v, grow)

        def fire_msg(c, b):
            pltpu.async_copy(
                msgs_hbm.at[cid, pl.ds(ebase + c * _SCH, _SCH)],
                mbuf[b], msem[b])
            pltpu.async_copy(rec_hbm.at[pl.ds(ebase + c * _SCH, _SCH)],
                             idxb[b], msem[b])

        def wait_msg(b):
            pltpu.make_async_copy(
                msgs_hbm.at[cid, pl.ds(0, _SCH)], mbuf[b], msem[b]).wait()
            pltpu.make_async_copy(rec_hbm.at[pl.ds(0, _SCH)], idxb[b],
                                  msem[b]).wait()

        def fire_sc(b):
            pltpu.async_copy(mbuf[b], shared.at[cidx[b]], ssem[b], add=True)

        def wait_sc(b):
            pltpu.make_async_copy(mbuf[b], shared.at[cidx[b]],
                                  ssem[b]).wait()

        def turn(c, b):
            wait_msg(b)
            build_cidx(b)
            fire_sc(b)
            b2 = (b + 2) % _NBUF

            @pl.when(c >= 2)
            def _():
                wait_sc(b2)

            @pl.when(c + 2 < _SNCH)
            def _():
                fire_msg(c + 2, b2)

        fire_msg(0, 0)
        fire_msg(1, 1)

        def round_(g, carry):
            for b in range(_NBUF):
                turn(_NBUF * g + b, b)
            return carry

        lax.fori_loop(0, _SNCH // _NBUF, round_, 0)
        turn(_SNCH - 2, 0)
        turn(_SNCH - 1, 1)
        wait_sc(0)
        wait_sc(1)

        plsc.subcore_barrier()

        @pl.when(cid == 0)
        def _():
            pltpu.sync_copy(shared.at[pl.ds(row0, _NPT)],
                            outx_hbm.at[pl.ds(nbase + row0, _NPT)])

        @pl.when(cid == 1)
        def _():
            pltpu.sync_copy(shared.at[pl.ds(row0, _NPT)],
                            outp_hbm.at[pl.ds(nbase + row0, _NPT)])

        @pl.when(sid == _NS - 1)
        def _():
            @pl.when(cid == 0)
            def _():
                pltpu.sync_copy(shared.at[pl.ds(tail0, _NTAIL)],
                                outx_hbm.at[pl.ds(nbase + tail0, _NTAIL)])

            @pl.when(cid == 1)
            def _():
                pltpu.sync_copy(shared.at[pl.ds(tail0, _NTAIL)],
                                outp_hbm.at[pl.ds(nbase + tail0, _NTAIL)])

        plsc.subcore_barrier()


_scatter = pl.kernel(
    _scatter_body,
    out_type=[jax.ShapeDtypeStruct((_N, _H), jnp.float32),
              jax.ShapeDtypeStruct((_N, _H), jnp.float32)],
    mesh=plsc.VectorSubcoreMesh(core_axis_name="c", subcore_axis_name="s"),
    compiler_params=pltpu.CompilerParams(needs_layout_passes=False),
    scratch_types=[
        pltpu.VMEM((_SCH,), jnp.int32),
        pltpu.VMEM((_SCH,), jnp.int32),
        pltpu.VMEM((_SCH,), jnp.int32),
        pltpu.VMEM((_SCH,), jnp.int32),
        pltpu.VMEM((_SCH,), jnp.int32),
        pltpu.VMEM((_SCH,), jnp.int32),
        pltpu.VMEM((_SCH,), jnp.int32),
        pltpu.VMEM((_SCH,), jnp.int32),
        pltpu.VMEM((_SCH, _H), jnp.float32),
        pltpu.VMEM((_SCH, _H), jnp.float32),
        pltpu.VMEM((_SCH, _H), jnp.float32),
        pltpu.VMEM((_SCH, _H), jnp.float32),
        pltpu.VMEM_SHARED((_SROWS, _H), jnp.float32),
        pltpu.SemaphoreType.DMA,
        pltpu.SemaphoreType.DMA,
        pltpu.SemaphoreType.DMA,
        pltpu.SemaphoreType.DMA,
        pltpu.SemaphoreType.DMA,
        pltpu.SemaphoreType.DMA,
        pltpu.SemaphoreType.DMA,
        pltpu.SemaphoreType.DMA,
    ],
)


def kernel(x, pos, pe, edge_index, W1, b1, W2, b2, Wp1, bp1, Wp2, bp2):
    send = edge_index[0]
    rec = edge_index[1]
    posT = pos.T.reshape(3 * _N)
    ts32, tr32 = _prep(x, pe, W1, b1.reshape(1, _H), Wp1,
                       bp1.reshape(1, _H))
    pre32, d2 = _gather(ts32, tr32, posT, send, rec)
    msgs = _mlp(pre32, d2[:, None], W2, b2.reshape(1, _H), Wp2,
                bp2.reshape(1, _H), W1[4 * _H:4 * _H + 1],
                Wp1[2 * _H:2 * _H + 1])
    update, update_pe = _scatter(msgs, x, pe, rec)
    return update, update_pe


# speedup vs baseline: 2.5199x; 1.0755x over previous
import jax
import jax.numpy as jnp
from jax import lax
from jax.experimental import pallas as pl
from jax.experimental.pallas import tpu as pltpu
from jax.experimental.pallas import tpu_sc as plsc

_N = 10000
_E = 320000
_H = 128

_NC = 2
_NS = 16
_NW = _NC * _NS


_PREP_BLK = 2000


def _dot(a, b):
    return lax.dot_general(a, b, (((1,), (0,)), ((), ())),
                           preferred_element_type=jnp.float32)


def _pack(a, b):
    ua = lax.bitcast_convert_type(
        a.astype(jnp.bfloat16).astype(jnp.float32), jnp.uint32)
    ub = lax.bitcast_convert_type(
        b.astype(jnp.bfloat16).astype(jnp.float32), jnp.uint32)
    w = (ua >> 16) | (ub & jnp.uint32(0xFFFF0000))
    return lax.bitcast_convert_type(w, jnp.int32)


def _prep_body(x_ref, pe_ref, W1_ref, b1_ref, Wp1_ref, bp1_ref,
               ts_ref, tr_ref):
    H = _H
    Q = _H // 2
    x = x_ref[...]
    pe = pe_ref[...]
    A = _dot(x, W1_ref[0:H]) + _dot(pe, W1_ref[H:2 * H]) + b1_ref[...]
    Ap = _dot(pe, Wp1_ref[0:H]) + bp1_ref[...]
    Bm = _dot(x, W1_ref[2 * H:3 * H]) + _dot(pe, W1_ref[3 * H:4 * H])
    Bp = _dot(pe, Wp1_ref[H:2 * H])
    ts_ref[:, 0:Q] = _pack(A[:, 0:Q], A[:, Q:H])
    ts_ref[:, Q:H] = _pack(Ap[:, 0:Q], Ap[:, Q:H])
    tr_ref[:, 0:Q] = _pack(Bm[:, 0:Q], Bm[:, Q:H])
    tr_ref[:, Q:H] = _pack(Bp[:, 0:Q], Bp[:, Q:H])


_prep = pl.pallas_call(
    _prep_body,
    grid=(_N // _PREP_BLK,),
    in_specs=[
        pl.BlockSpec((_PREP_BLK, _H), lambda i: (i, 0)),
        pl.BlockSpec((_PREP_BLK, _H), lambda i: (i, 0)),
        pl.BlockSpec((4 * _H + 1, _H), lambda i: (0, 0)),
        pl.BlockSpec((1, _H), lambda i: (0, 0)),
        pl.BlockSpec((2 * _H + 1, _H), lambda i: (0, 0)),
        pl.BlockSpec((1, _H), lambda i: (0, 0)),
    ],
    out_specs=[
        pl.BlockSpec((_PREP_BLK, _H), lambda i: (i, 0)),
        pl.BlockSpec((_PREP_BLK, _H), lambda i: (i, 0)),
    ],
    out_shape=[
        jax.ShapeDtypeStruct((_N, _H), jnp.int32),
        jax.ShapeDtypeStruct((_N, _H), jnp.int32),
    ],
)


_EPW = _E // _NW
_GCH = 80
_GNCH = _EPW // _GCH


def _gather_body(ts_hbm, tr_hbm, posT_hbm, send_hbm, rec_hbm,
                 pre_hbm, d2_hbm,
                 sidx_a, ridx_a, rs_a, rr_a,
                 sidx_b, ridx_b, rs_b, rr_b,
                 posT_v, d2_all,
                 isem_a, isem_b, g1sem_a, g2sem_a, g1sem_b, g2sem_b,
                 osem_a, osem_b):
    cid = lax.axis_index("c")
    sid = lax.axis_index("s")
    wid = sid * _NC + cid
    tbase = wid * _EPW
    pltpu.sync_copy(posT_hbm, posT_v)
    offy = jnp.full((16,), _N, jnp.int32)
    offz = jnp.full((16,), 2 * _N, jnp.int32)

    A = (sidx_a, ridx_a, rs_a, rr_a, isem_a, g1sem_a, g2sem_a, osem_a)
    B = (sidx_b, ridx_b, rs_b, rr_b, isem_b, g1sem_b, g2sem_b, osem_b)

    def fire_idx(c, S):
        sidx, ridx = S[0], S[1]
        base = tbase + c * _GCH
        pltpu.async_copy(send_hbm.at[pl.ds(base, _GCH)], sidx, S[4])
        pltpu.async_copy(rec_hbm.at[pl.ds(base, _GCH)], ridx, S[4])

    def wait_idx(S):
        pltpu.make_async_copy(send_hbm.at[pl.ds(0, _GCH)], S[0], S[4]).wait()
        pltpu.make_async_copy(rec_hbm.at[pl.ds(0, _GCH)], S[1], S[4]).wait()

    def fire_gathers(S):
        pltpu.async_copy(ts_hbm.at[S[0]], S[2], S[5])
        pltpu.async_copy(tr_hbm.at[S[1]], S[3], S[6])

    def wait_gathers(S):
        pltpu.make_async_copy(ts_hbm.at[S[0]], S[2], S[5]).wait()
        pltpu.make_async_copy(tr_hbm.at[S[1]], S[3], S[6]).wait()

    def fire_out(c, S):
        pltpu.async_copy(S[2], pre_hbm.at[pl.ds(tbase + c * _GCH, _GCH)],
                         S[7])

    def wait_out(S):
        pltpu.make_async_copy(S[2], pre_hbm.at[pl.ds(0, _GCH)], S[7]).wait()

    def d2_compute(c, S):
        sidx, ridx = S[0], S[1]
        for g in range(_GCH // 16):
            is_ = sidx[pl.ds(g * 16, 16)]
            ir_ = ridx[pl.ds(g * 16, 16)]
            dx = (plsc.load_gather(posT_v, [is_])
                  - plsc.load_gather(posT_v, [ir_]))
            dy = (plsc.load_gather(posT_v, [is_ + offy])
                  - plsc.load_gather(posT_v, [ir_ + offy]))
            dz = (plsc.load_gather(posT_v, [is_ + offz])
                  - plsc.load_gather(posT_v, [ir_ + offz]))
            d2_all[pl.ds(c * _GCH + g * 16, 16)] = dx * dx + dy * dy + dz * dz

    def adds(S):
        rs, rr = S[2], S[3]

        def row(i, carry):
            for j in range(_H // 16):
                sl = pl.ds(j * 16, 16)
                a = plsc.bitcast(rs[i, sl], jnp.bfloat16)
                b = plsc.bitcast(rr[i, sl], jnp.bfloat16)
                rs[i, sl] = plsc.bitcast(a + b, jnp.int32)
            return carry

        lax.fori_loop(0, _GCH, row, 0)

    def half(c, S, Sp, fire_next=True):
        wait_gathers(S)
        d2_compute(c, S)

        @pl.when(c + 2 < _GNCH)
        def _():
            fire_idx(c + 2, S)

        if fire_next:
            wait_idx(Sp)

            @pl.when(c >= 1)
            def _():
                wait_out(Sp)

            fire_gathers(Sp)
        adds(S)
        fire_out(c, S)

    fire_idx(0, A)
    fire_idx(1, B)
    wait_idx(A)
    fire_gathers(A)

    def round_(g, carry):
        half(2 * g, A, B)
        half(2 * g + 1, B, A)
        return carry

    lax.fori_loop(0, _GNCH // 2, round_, 0)
    half(_GNCH - 1, A, B, fire_next=False)
    wait_out(B)
    wait_out(A)
    pltpu.sync_copy(d2_all, d2_hbm.at[pl.ds(tbase, _EPW)])


_gather = pl.kernel(
    _gather_body,
    out_type=[
        jax.ShapeDtypeStruct((_E, _H), jnp.int32),
        jax.ShapeDtypeStruct((_E,), jnp.float32),
    ],
    mesh=plsc.VectorSubcoreMesh(core_axis_name="c", subcore_axis_name="s"),
    compiler_params=pltpu.CompilerParams(needs_layout_passes=False),
    scratch_types=[
        pltpu.VMEM((_GCH,), jnp.int32),
        pltpu.VMEM((_GCH,), jnp.int32),
        pltpu.VMEM((_GCH, _H), jnp.int32),
        pltpu.VMEM((_GCH, _H), jnp.int32),
        pltpu.VMEM((_GCH,), jnp.int32),
        pltpu.VMEM((_GCH,), jnp.int32),
        pltpu.VMEM((_GCH, _H), jnp.int32),
        pltpu.VMEM((_GCH, _H), jnp.int32),
        pltpu.VMEM((3 * _N,), jnp.float32),
        pltpu.VMEM((_EPW,), jnp.float32),
        pltpu.SemaphoreType.DMA,
        pltpu.SemaphoreType.DMA,
        pltpu.SemaphoreType.DMA,
        pltpu.SemaphoreType.DMA,
        pltpu.SemaphoreType.DMA,
        pltpu.SemaphoreType.DMA,
        pltpu.SemaphoreType.DMA,
        pltpu.SemaphoreType.DMA,
    ],
)


_MLP_BLK = 2000


def _unpack(w):
    u = lax.bitcast_convert_type(w, jnp.uint32)
    lo = lax.bitcast_convert_type(u << 16, jnp.float32)
    hi = lax.bitcast_convert_type(u & jnp.uint32(0xFFFF0000), jnp.float32)
    return jnp.concatenate([lo, hi], axis=1)


def _silu(x):
    return x * (0.5 * jnp.tanh(0.5 * x) + 0.5)


def _mlp_body(pre_ref, d2_ref, W2_ref, b2_ref, Wp2_ref, bp2_ref,
              wl_ref, wpl_ref, out_ref):
    H = _H
    Q = _H // 2
    dist = jnp.sqrt(d2_ref[...])
    h = _unpack(pre_ref[:, 0:Q]) + dist * wl_ref[...]
    h = _silu(h)
    m = _dot(h, W2_ref[...]) + b2_ref[...]
    out_ref[0] = _silu(m)
    hp = jnp.tanh(_unpack(pre_ref[:, Q:H]) + dist * wpl_ref[...])
    out_ref[1] = jnp.tanh(_dot(hp, Wp2_ref[...]) + bp2_ref[...])


_mlp = pl.pallas_call(
    _mlp_body,
    grid=(_E // _MLP_BLK,),
    in_specs=[
        pl.BlockSpec((_MLP_BLK, _H), lambda i: (i, 0)),
        pl.BlockSpec((_MLP_BLK, 1), lambda i: (i, 0)),
        pl.BlockSpec((_H, _H), lambda i: (0, 0)),
        pl.BlockSpec((1, _H), lambda i: (0, 0)),
        pl.BlockSpec((_H, _H), lambda i: (0, 0)),
        pl.BlockSpec((1, _H), lambda i: (0, 0)),
        pl.BlockSpec((1, _H), lambda i: (0, 0)),
        pl.BlockSpec((1, _H), lambda i: (0, 0)),
    ],
    out_specs=pl.BlockSpec((2, _MLP_BLK, _H), lambda i: (0, i, 0)),
    out_shape=jax.ShapeDtypeStruct((2, _E, _H), jnp.float32),
)


_NP = 2
_NH = _N // _NP
_SROWS = _NH + 8
_NPT = 312
_NTAIL = _NH - _NS * _NPT
_EPT = _E // _NS
_SCH = 80
_SNCH = _EPT // _SCH
_NBUF = 6


def _scatter_body(msgs_hbm, x_hbm, pe_hbm, rec_hbm, outx_hbm, outp_hbm,
                  idxb0, idxb1, idxb2, idxb3, idxb4, idxb5,
                  cidx0, cidx1, cidx2, cidx3, cidx4, cidx5,
                  mbuf0, mbuf1, mbuf2, mbuf3, mbuf4, mbuf5,
                  shared,
                  msem0, msem1, msem2, msem3, msem4, msem5,
                  ssem0, ssem1, ssem2, ssem3, ssem4, ssem5):
    cid = lax.axis_index("c")
    sid = lax.axis_index("s")
    row0 = pl.multiple_of(sid * _NPT, 8)
    tail0 = _NS * _NPT
    ebase = sid * _EPT
    idxb = (idxb0, idxb1, idxb2, idxb3, idxb4, idxb5)
    cidx = (cidx0, cidx1, cidx2, cidx3, cidx4, cidx5)
    mbuf = (mbuf0, mbuf1, mbuf2, mbuf3, mbuf4, mbuf5)
    msem = (msem0, msem1, msem2, msem3, msem4, msem5)
    ssem = (ssem0, ssem1, ssem2, ssem3, ssem4, ssem5)

    for p in range(_NP):
        nbase = p * _NH

        @pl.when(cid == 0)
        def _():
            pltpu.sync_copy(x_hbm.at[pl.ds(nbase + row0, _NPT)],
                            shared.at[pl.ds(row0, _NPT)])

        @pl.when(cid == 1)
        def _():
            pltpu.sync_copy(pe_hbm.at[pl.ds(nbase + row0, _NPT)],
                            shared.at[pl.ds(row0, _NPT)])

        @pl.when(sid == _NS - 1)
        def _():
            @pl.when(cid == 0)
            def _():
                pltpu.sync_copy(x_hbm.at[pl.ds(nbase + tail0, _NTAIL)],
                                shared.at[pl.ds(tail0, _NTAIL)])

            @pl.when(cid == 1)
            def _():
                pltpu.sync_copy(pe_hbm.at[pl.ds(nbase + tail0, _NTAIL)],
                                shared.at[pl.ds(tail0, _NTAIL)])

        plsc.subcore_barrier()

        def build_cidx(b):
            for g in range(_SCH // 16):
                v = idxb[b][pl.ds(g * 16, 16)] - (p * _NH)
                ok = (v >= 0) & (v < _NH)
                grow = (lax.iota(jnp.int32, 16) & 7) + _NH
                cidx[b][pl.ds(g * 16, 16)] = jnp.where(ok, v, grow)

        def fire_msg(c, b):
            pltpu.async_copy(
                msgs_hbm.at[cid, pl.ds(ebase + c * _SCH, _SCH)],
                mbuf[b], msem[b])
            pltpu.async_copy(rec_hbm.at[pl.ds(ebase + c * _SCH, _SCH)],
                             idxb[b], msem[b])

        def wait_msg(b):
            pltpu.make_async_copy(
                msgs_hbm.at[cid, pl.ds(0, _SCH)], mbuf[b], msem[b]).wait()
            pltpu.make_async_copy(rec_hbm.at[pl.ds(0, _SCH)], idxb[b],
                                  msem[b]).wait()

        def fire_sc(b):
            pltpu.async_copy(mbuf[b], shared.at[cidx[b]], ssem[b], add=True)

        def wait_sc(b):
            pltpu.make_async_copy(mbuf[b], shared.at[cidx[b]],
                                  ssem[b]).wait()

        def turn(c, b):
            wait_msg(b)
            build_cidx(b)
            fire_sc(b)
            b4 = (b + 4) % _NBUF

            @pl.when(c >= 2)
            def _():
                wait_sc(b4)

            @pl.when(c + 4 < _SNCH)
            def _():
                fire_msg(c + 4, b4)

        for cpro in range(4):
            fire_msg(cpro, cpro)

        def round_(g, carry):
            for b in range(_NBUF):
                turn(_NBUF * g + b, b)
            return carry

        lax.fori_loop(0, _SNCH // _NBUF, round_, 0)
        for ctail in range(_NBUF * (_SNCH // _NBUF), _SNCH):
            turn(ctail, ctail % _NBUF)
        wait_sc((_SNCH - 2) % _NBUF)
        wait_sc((_SNCH - 1) % _NBUF)

        plsc.subcore_barrier()

        @pl.when(cid == 0)
        def _():
            pltpu.sync_copy(shared.at[pl.ds(row0, _NPT)],
                            outx_hbm.at[pl.ds(nbase + row0, _NPT)])

        @pl.when(cid == 1)
        def _():
            pltpu.sync_copy(shared.at[pl.ds(row0, _NPT)],
                            outp_hbm.at[pl.ds(nbase + row0, _NPT)])

        @pl.when(sid == _NS - 1)
        def _():
            @pl.when(cid == 0)
            def _():
                pltpu.sync_copy(shared.at[pl.ds(tail0, _NTAIL)],
                                outx_hbm.at[pl.ds(nbase + tail0, _NTAIL)])

            @pl.when(cid == 1)
            def _():
                pltpu.sync_copy(shared.at[pl.ds(tail0, _NTAIL)],
                                outp_hbm.at[pl.ds(nbase + tail0, _NTAIL)])

        plsc.subcore_barrier()


_scatter = pl.kernel(
    _scatter_body,
    out_type=[jax.ShapeDtypeStruct((_N, _H), jnp.float32),
              jax.ShapeDtypeStruct((_N, _H), jnp.float32)],
    mesh=plsc.VectorSubcoreMesh(core_axis_name="c", subcore_axis_name="s"),
    compiler_params=pltpu.CompilerParams(needs_layout_passes=False),
    scratch_types=[
        pltpu.VMEM((_SCH,), jnp.int32),
        pltpu.VMEM((_SCH,), jnp.int32),
        pltpu.VMEM((_SCH,), jnp.int32),
        pltpu.VMEM((_SCH,), jnp.int32),
        pltpu.VMEM((_SCH,), jnp.int32),
        pltpu.VMEM((_SCH,), jnp.int32),
        pltpu.VMEM((_SCH,), jnp.int32),
        pltpu.VMEM((_SCH,), jnp.int32),
        pltpu.VMEM((_SCH,), jnp.int32),
        pltpu.VMEM((_SCH,), jnp.int32),
        pltpu.VMEM((_SCH,), jnp.int32),
        pltpu.VMEM((_SCH,), jnp.int32),
        pltpu.VMEM((_SCH, _H), jnp.float32),
        pltpu.VMEM((_SCH, _H), jnp.float32),
        pltpu.VMEM((_SCH, _H), jnp.float32),
        pltpu.VMEM((_SCH, _H), jnp.float32),
        pltpu.VMEM((_SCH, _H), jnp.float32),
        pltpu.VMEM((_SCH, _H), jnp.float32),
        pltpu.VMEM_SHARED((_SROWS, _H), jnp.float32),
        pltpu.SemaphoreType.DMA,
        pltpu.SemaphoreType.DMA,
        pltpu.SemaphoreType.DMA,
        pltpu.SemaphoreType.DMA,
        pltpu.SemaphoreType.DMA,
        pltpu.SemaphoreType.DMA,
        pltpu.SemaphoreType.DMA,
        pltpu.SemaphoreType.DMA,
        pltpu.SemaphoreType.DMA,
        pltpu.SemaphoreType.DMA,
        pltpu.SemaphoreType.DMA,
        pltpu.SemaphoreType.DMA,
    ],
)


def kernel(x, pos, pe, edge_index, W1, b1, W2, b2, Wp1, bp1, Wp2, bp2):
    send = edge_index[0]
    rec = edge_index[1]
    posT = pos.T.reshape(3 * _N)
    ts32, tr32 = _prep(x, pe, W1, b1.reshape(1, _H), Wp1,
                       bp1.reshape(1, _H))
    pre32, d2 = _gather(ts32, tr32, posT, send, rec)
    msgs = _mlp(pre32, d2[:, None], W2, b2.reshape(1, _H), Wp2,
                bp2.reshape(1, _H), W1[4 * _H:4 * _H + 1],
                Wp1[2 * _H:2 * _H + 1])
    update, update_pe = _scatter(msgs, x, pe, rec)
    return update, update_pe


# speedup vs baseline: 2.5231x; 1.0012x over previous
import jax
import jax.numpy as jnp
from jax import lax
from jax.experimental import pallas as pl
from jax.experimental.pallas import tpu as pltpu
from jax.experimental.pallas import tpu_sc as plsc

_N = 10000
_E = 320000
_H = 128

_NC = 2
_NS = 16
_NW = _NC * _NS


_PREP_BLK = 2000


def _dot(a, b):
    return lax.dot_general(a, b, (((1,), (0,)), ((), ())),
                           preferred_element_type=jnp.float32)


def _pack(a, b):
    ua = lax.bitcast_convert_type(
        a.astype(jnp.bfloat16).astype(jnp.float32), jnp.uint32)
    ub = lax.bitcast_convert_type(
        b.astype(jnp.bfloat16).astype(jnp.float32), jnp.uint32)
    w = (ua >> 16) | (ub & jnp.uint32(0xFFFF0000))
    return lax.bitcast_convert_type(w, jnp.int32)


def _prep_body(x_ref, pe_ref, W1_ref, b1_ref, Wp1_ref, bp1_ref,
               ts_ref, tr_ref):
    H = _H
    Q = _H // 2
    x = x_ref[...]
    pe = pe_ref[...]
    A = _dot(x, W1_ref[0:H]) + _dot(pe, W1_ref[H:2 * H]) + b1_ref[...]
    Ap = _dot(pe, Wp1_ref[0:H]) + bp1_ref[...]
    Bm = _dot(x, W1_ref[2 * H:3 * H]) + _dot(pe, W1_ref[3 * H:4 * H])
    Bp = _dot(pe, Wp1_ref[H:2 * H])
    ts_ref[:, 0:Q] = _pack(A[:, 0:Q], A[:, Q:H])
    ts_ref[:, Q:H] = _pack(Ap[:, 0:Q], Ap[:, Q:H])
    tr_ref[:, 0:Q] = _pack(Bm[:, 0:Q], Bm[:, Q:H])
    tr_ref[:, Q:H] = _pack(Bp[:, 0:Q], Bp[:, Q:H])


_prep = pl.pallas_call(
    _prep_body,
    grid=(_N // _PREP_BLK,),
    in_specs=[
        pl.BlockSpec((_PREP_BLK, _H), lambda i: (i, 0)),
        pl.BlockSpec((_PREP_BLK, _H), lambda i: (i, 0)),
        pl.BlockSpec((4 * _H + 1, _H), lambda i: (0, 0)),
        pl.BlockSpec((1, _H), lambda i: (0, 0)),
        pl.BlockSpec((2 * _H + 1, _H), lambda i: (0, 0)),
        pl.BlockSpec((1, _H), lambda i: (0, 0)),
    ],
    out_specs=[
        pl.BlockSpec((_PREP_BLK, _H), lambda i: (i, 0)),
        pl.BlockSpec((_PREP_BLK, _H), lambda i: (i, 0)),
    ],
    out_shape=[
        jax.ShapeDtypeStruct((_N, _H), jnp.int32),
        jax.ShapeDtypeStruct((_N, _H), jnp.int32),
    ],
)


_EPW = _E // _NW
_GCH = 80
_GNCH = _EPW // _GCH


def _gather_body(ts_hbm, tr_hbm, posT_hbm, send_hbm, rec_hbm,
                 pre_hbm, d2_hbm,
                 sidx_a, ridx_a, rs_a, rr_a,
                 sidx_b, ridx_b, rs_b, rr_b,
                 posT_v, d2_all,
                 isem_a, isem_b, g1sem_a, g2sem_a, g1sem_b, g2sem_b,
                 osem_a, osem_b):
    cid = lax.axis_index("c")
    sid = lax.axis_index("s")
    wid = sid * _NC + cid
    tbase = wid * _EPW
    pltpu.sync_copy(posT_hbm, posT_v)
    offy = jnp.full((16,), _N, jnp.int32)
    offz = jnp.full((16,), 2 * _N, jnp.int32)

    A = (sidx_a, ridx_a, rs_a, rr_a, isem_a, g1sem_a, g2sem_a, osem_a)
    B = (sidx_b, ridx_b, rs_b, rr_b, isem_b, g1sem_b, g2sem_b, osem_b)

    def fire_idx(c, S):
        sidx, ridx = S[0], S[1]
        base = tbase + c * _GCH
        pltpu.async_copy(send_hbm.at[pl.ds(base, _GCH)], sidx, S[4])
        pltpu.async_copy(rec_hbm.at[pl.ds(base, _GCH)], ridx, S[4])

    def wait_idx(S):
        pltpu.make_async_copy(send_hbm.at[pl.ds(0, _GCH)], S[0], S[4]).wait()
        pltpu.make_async_copy(rec_hbm.at[pl.ds(0, _GCH)], S[1], S[4]).wait()

    def fire_gathers(S):
        pltpu.async_copy(ts_hbm.at[S[0]], S[2], S[5])
        pltpu.async_copy(tr_hbm.at[S[1]], S[3], S[6])

    def wait_gathers(S):
        pltpu.make_async_copy(ts_hbm.at[S[0]], S[2], S[5]).wait()
        pltpu.make_async_copy(tr_hbm.at[S[1]], S[3], S[6]).wait()

    def fire_out(c, S):
        pltpu.async_copy(S[2], pre_hbm.at[pl.ds(tbase + c * _GCH, _GCH)],
                         S[7])

    def wait_out(S):
        pltpu.make_async_copy(S[2], pre_hbm.at[pl.ds(0, _GCH)], S[7]).wait()

    def d2_compute(c, S):
        sidx, ridx = S[0], S[1]
        for g in range(_GCH // 16):
            is_ = sidx[pl.ds(g * 16, 16)]
            ir_ = ridx[pl.ds(g * 16, 16)]
            dx = (plsc.load_gather(posT_v, [is_])
                  - plsc.load_gather(posT_v, [ir_]))
            dy = (plsc.load_gather(posT_v, [is_ + offy])
                  - plsc.load_gather(posT_v, [ir_ + offy]))
            dz = (plsc.load_gather(posT_v, [is_ + offz])
                  - plsc.load_gather(posT_v, [ir_ + offz]))
            d2_all[pl.ds(c * _GCH + g * 16, 16)] = dx * dx + dy * dy + dz * dz

    def adds(S):
        rs, rr = S[2], S[3]

        def row(i, carry):
            for j in range(_H // 16):
                sl = pl.ds(j * 16, 16)
                a = plsc.bitcast(rs[i, sl], jnp.bfloat16)
                b = plsc.bitcast(rr[i, sl], jnp.bfloat16)
                rs[i, sl] = plsc.bitcast(a + b, jnp.int32)
            return carry

        lax.fori_loop(0, _GCH, row, 0)

    def half(c, S, Sp, fire_next=True):
        wait_gathers(S)
        d2_compute(c, S)

        @pl.when(c + 2 < _GNCH)
        def _():
            fire_idx(c + 2, S)

        if fire_next:
            wait_idx(Sp)

            @pl.when(c >= 1)
            def _():
                wait_out(Sp)

            fire_gathers(Sp)
        adds(S)
        fire_out(c, S)

    fire_idx(0, A)
    fire_idx(1, B)
    wait_idx(A)
    fire_gathers(A)

    def round_(g, carry):
        half(2 * g, A, B)
        half(2 * g + 1, B, A)
        return carry

    lax.fori_loop(0, _GNCH // 2, round_, 0)
    half(_GNCH - 1, A, B, fire_next=False)
    wait_out(B)
    wait_out(A)
    pltpu.sync_copy(d2_all, d2_hbm.at[pl.ds(tbase, _EPW)])


_gather = pl.kernel(
    _gather_body,
    out_type=[
        jax.ShapeDtypeStruct((_E, _H), jnp.int32),
        jax.ShapeDtypeStruct((_E,), jnp.float32),
    ],
    mesh=plsc.VectorSubcoreMesh(core_axis_name="c", subcore_axis_name="s"),
    compiler_params=pltpu.CompilerParams(needs_layout_passes=False),
    scratch_types=[
        pltpu.VMEM((_GCH,), jnp.int32),
        pltpu.VMEM((_GCH,), jnp.int32),
        pltpu.VMEM((_GCH, _H), jnp.int32),
        pltpu.VMEM((_GCH, _H), jnp.int32),
        pltpu.VMEM((_GCH,), jnp.int32),
        pltpu.VMEM((_GCH,), jnp.int32),
        pltpu.VMEM((_GCH, _H), jnp.int32),
        pltpu.VMEM((_GCH, _H), jnp.int32),
        pltpu.VMEM((3 * _N,), jnp.float32),
        pltpu.VMEM((_EPW,), jnp.float32),
        pltpu.SemaphoreType.DMA,
        pltpu.SemaphoreType.DMA,
        pltpu.SemaphoreType.DMA,
        pltpu.SemaphoreType.DMA,
        pltpu.SemaphoreType.DMA,
        pltpu.SemaphoreType.DMA,
        pltpu.SemaphoreType.DMA,
        pltpu.SemaphoreType.DMA,
    ],
)


_MLP_BLK = 2000


def _unpack(w):
    u = lax.bitcast_convert_type(w, jnp.uint32)
    lo = lax.bitcast_convert_type(u << 16, jnp.float32)
    hi = lax.bitcast_convert_type(u & jnp.uint32(0xFFFF0000), jnp.float32)
    return jnp.concatenate([lo, hi], axis=1)


def _silu(x):
    return x * (0.5 * jnp.tanh(0.5 * x) + 0.5)


def _mlp_body(pre_ref, d2_ref, W2_ref, b2_ref, Wp2_ref, bp2_ref,
              wl_ref, wpl_ref, out_ref):
    H = _H
    Q = _H // 2
    dist = jnp.sqrt(d2_ref[...])
    h = _unpack(pre_ref[:, 0:Q]) + dist * wl_ref[...]
    h = _silu(h)
    m = _dot(h, W2_ref[...]) + b2_ref[...]
    out_ref[0] = _silu(m)
    hp = jnp.tanh(_unpack(pre_ref[:, Q:H]) + dist * wpl_ref[...])
    out_ref[1] = jnp.tanh(_dot(hp, Wp2_ref[...]) + bp2_ref[...])


_mlp = pl.pallas_call(
    _mlp_body,
    grid=(_E // _MLP_BLK,),
    in_specs=[
        pl.BlockSpec((_MLP_BLK, _H), lambda i: (i, 0)),
        pl.BlockSpec((_MLP_BLK, 1), lambda i: (i, 0)),
        pl.BlockSpec((_H, _H), lambda i: (0, 0)),
        pl.BlockSpec((1, _H), lambda i: (0, 0)),
        pl.BlockSpec((_H, _H), lambda i: (0, 0)),
        pl.BlockSpec((1, _H), lambda i: (0, 0)),
        pl.BlockSpec((1, _H), lambda i: (0, 0)),
        pl.BlockSpec((1, _H), lambda i: (0, 0)),
    ],
    out_specs=pl.BlockSpec((2, _MLP_BLK, _H), lambda i: (0, i, 0)),
    out_shape=jax.ShapeDtypeStruct((2, _E, _H), jnp.float32),
)


_NP = 2
_NH = _N // _NP
_SROWS = _NH + 8
_NPT = 312
_NTAIL = _NH - _NS * _NPT
_EPT = _E // _NS
_SCH = 80
_SNCH = _EPT // _SCH
_NBUF = 8


def _scatter_body(msgs_hbm, x_hbm, pe_hbm, rec_hbm, outx_hbm, outp_hbm,
                  idxb0, idxb1, idxb2, idxb3, idxb4, idxb5, idxb6, idxb7,
                  cidx0, cidx1, cidx2, cidx3, cidx4, cidx5, cidx6, cidx7,
                  mbuf0, mbuf1, mbuf2, mbuf3, mbuf4, mbuf5, mbuf6, mbuf7,
                  shared,
                  msem0, msem1, msem2, msem3, msem4, msem5, msem6, msem7,
                  ssem0, ssem1, ssem2, ssem3, ssem4, ssem5, ssem6, ssem7):
    cid = lax.axis_index("c")
    sid = lax.axis_index("s")
    row0 = pl.multiple_of(sid * _NPT, 8)
    tail0 = _NS * _NPT
    ebase = sid * _EPT
    idxb = (idxb0, idxb1, idxb2, idxb3, idxb4, idxb5, idxb6, idxb7)
    cidx = (cidx0, cidx1, cidx2, cidx3, cidx4, cidx5, cidx6, cidx7)
    mbuf = (mbuf0, mbuf1, mbuf2, mbuf3, mbuf4, mbuf5, mbuf6, mbuf7)
    msem = (msem0, msem1, msem2, msem3, msem4, msem5, msem6, msem7)
    ssem = (ssem0, ssem1, ssem2, ssem3, ssem4, ssem5, ssem6, ssem7)

    for p in range(_NP):
        nbase = p * _NH

        @pl.when(cid == 0)
        def _():
            pltpu.sync_copy(x_hbm.at[pl.ds(nbase + row0, _NPT)],
                            shared.at[pl.ds(row0, _NPT)])

        @pl.when(cid == 1)
        def _():
            pltpu.sync_copy(pe_hbm.at[pl.ds(nbase + row0, _NPT)],
                            shared.at[pl.ds(row0, _NPT)])

        @pl.when(sid == _NS - 1)
        def _():
            @pl.when(cid == 0)
            def _():
                pltpu.sync_copy(x_hbm.at[pl.ds(nbase + tail0, _NTAIL)],
                                shared.at[pl.ds(tail0, _NTAIL)])

            @pl.when(cid == 1)
            def _():
                pltpu.sync_copy(pe_hbm.at[pl.ds(nbase + tail0, _NTAIL)],
                                shared.at[pl.ds(tail0, _NTAIL)])

        plsc.subcore_barrier()

        def build_cidx(b):
            for g in range(_SCH // 16):
                v = idxb[b][pl.ds(g * 16, 16)] - (p * _NH)
                ok = (v >= 0) & (v < _NH)
                grow = (lax.iota(jnp.int32, 16) & 7) + _NH
                cidx[b][pl.ds(g * 16, 16)] = jnp.where(ok, v, grow)

        def fire_msg(c, b):
            pltpu.async_copy(
                msgs_hbm.at[cid, pl.ds(ebase + c * _SCH, _SCH)],
                mbuf[b], msem[b])
            pltpu.async_copy(rec_hbm.at[pl.ds(ebase + c * _SCH, _SCH)],
                             idxb[b], msem[b])

        def wait_msg(b):
            pltpu.make_async_copy(
                msgs_hbm.at[cid, pl.ds(0, _SCH)], mbuf[b], msem[b]).wait()
            pltpu.make_async_copy(rec_hbm.at[pl.ds(0, _SCH)], idxb[b],
                                  msem[b]).wait()

        def fire_sc(b):
            pltpu.async_copy(mbuf[b], shared.at[cidx[b]], ssem[b], add=True)

        def wait_sc(b):
            pltpu.make_async_copy(mbuf[b], shared.at[cidx[b]],
                                  ssem[b]).wait()

        def turn(c, b):
            wait_msg(b)
            build_cidx(b)
            fire_sc(b)
            bn = (b + _NBUF - 2) % _NBUF

            @pl.when(c >= 2)
            def _():
                wait_sc(bn)

            @pl.when(c + _NBUF - 2 < _SNCH)
            def _():
                fire_msg(c + _NBUF - 2, bn)

        for cpro in range(_NBUF - 2):
            fire_msg(cpro, cpro)

        def round_(g, carry):
            for b in range(_NBUF):
                turn(_NBUF * g + b, b)
            return carry

        lax.fori_loop(0, _SNCH // _NBUF, round_, 0)
        for ctail in range(_NBUF * (_SNCH // _NBUF), _SNCH):
            turn(ctail, ctail % _NBUF)
        wait_sc((_SNCH - 2) % _NBUF)
        wait_sc((_SNCH - 1) % _NBUF)

        plsc.subcore_barrier()

        @pl.when(cid == 0)
        def _():
            pltpu.sync_copy(shared.at[pl.ds(row0, _NPT)],
                            outx_hbm.at[pl.ds(nbase + row0, _NPT)])

        @pl.when(cid == 1)
        def _():
            pltpu.sync_copy(shared.at[pl.ds(row0, _NPT)],
                            outp_hbm.at[pl.ds(nbase + row0, _NPT)])

        @pl.when(sid == _NS - 1)
        def _():
            @pl.when(cid == 0)
            def _():
                pltpu.sync_copy(shared.at[pl.ds(tail0, _NTAIL)],
                                outx_hbm.at[pl.ds(nbase + tail0, _NTAIL)])

            @pl.when(cid == 1)
            def _():
                pltpu.sync_copy(shared.at[pl.ds(tail0, _NTAIL)],
                                outp_hbm.at[pl.ds(nbase + tail0, _NTAIL)])

        plsc.subcore_barrier()


_scatter = pl.kernel(
    _scatter_body,
    out_type=[jax.ShapeDtypeStruct((_N, _H), jnp.float32),
              jax.ShapeDtypeStruct((_N, _H), jnp.float32)],
    mesh=plsc.VectorSubcoreMesh(core_axis_name="c", subcore_axis_name="s"),
    compiler_params=pltpu.CompilerParams(needs_layout_passes=False),
    scratch_types=[
        pltpu.VMEM((_SCH,), jnp.int32),
        pltpu.VMEM((_SCH,), jnp.int32),
        pltpu.VMEM((_SCH,), jnp.int32),
        pltpu.VMEM((_SCH,), jnp.int32),
        pltpu.VMEM((_SCH,), jnp.int32),
        pltpu.VMEM((_SCH,), jnp.int32),
        pltpu.VMEM((_SCH,), jnp.int32),
        pltpu.VMEM((_SCH,), jnp.int32),
        pltpu.VMEM((_SCH,), jnp.int32),
        pltpu.VMEM((_SCH,), jnp.int32),
        pltpu.VMEM((_SCH,), jnp.int32),
        pltpu.VMEM((_SCH,), jnp.int32),
        pltpu.VMEM((_SCH,), jnp.int32),
        pltpu.VMEM((_SCH,), jnp.int32),
        pltpu.VMEM((_SCH,), jnp.int32),
        pltpu.VMEM((_SCH,), jnp.int32),
        pltpu.VMEM((_SCH, _H), jnp.float32),
        pltpu.VMEM((_SCH, _H), jnp.float32),
        pltpu.VMEM((_SCH, _H), jnp.float32),
        pltpu.VMEM((_SCH, _H), jnp.float32),
        pltpu.VMEM((_SCH, _H), jnp.float32),
        pltpu.VMEM((_SCH, _H), jnp.float32),
        pltpu.VMEM((_SCH, _H), jnp.float32),
        pltpu.VMEM((_SCH, _H), jnp.float32),
        pltpu.VMEM_SHARED((_SROWS, _H), jnp.float32),
        pltpu.SemaphoreType.DMA,
        pltpu.SemaphoreType.DMA,
        pltpu.SemaphoreType.DMA,
        pltpu.SemaphoreType.DMA,
        pltpu.SemaphoreType.DMA,
        pltpu.SemaphoreType.DMA,
        pltpu.SemaphoreType.DMA,
        pltpu.SemaphoreType.DMA,
        pltpu.SemaphoreType.DMA,
        pltpu.SemaphoreType.DMA,
        pltpu.SemaphoreType.DMA,
        pltpu.SemaphoreType.DMA,
        pltpu.SemaphoreType.DMA,
        pltpu.SemaphoreType.DMA,
        pltpu.SemaphoreType.DMA,
        pltpu.SemaphoreType.DMA,
    ],
)


def kernel(x, pos, pe, edge_index, W1, b1, W2, b2, Wp1, bp1, Wp2, bp2):
    send = edge_index[0]
    rec = edge_index[1]
    posT = pos.T.reshape(3 * _N)
    ts32, tr32 = _prep(x, pe, W1, b1.reshape(1, _H), Wp1,
                       bp1.reshape(1, _H))
    pre32, d2 = _gather(ts32, tr32, posT, send, rec)
    msgs = _mlp(pre32, d2[:, None], W2, b2.reshape(1, _H), Wp2,
                bp2.reshape(1, _H), W1[4 * _H:4 * _H + 1],
                Wp1[2 * _H:2 * _H + 1])
    update, update_pe = _scatter(msgs, x, pe, rec)
    return update, update_pe
